# Initial kernel scaffold; baseline (speedup 1.0000x reference)
import jax
import jax.numpy as jnp
from jax.experimental import pallas as pl


def kernel(input, mask_tight, mask_rough, style_his):
    raise NotImplementedError("write your pallas kernel here")


# speedup vs baseline: 825.2691x; 825.2691x over previous
import jax
import jax.numpy as jnp
from jax import lax
from jax.experimental import pallas as pl
from jax.experimental.pallas import tpu as pltpu

BINS_ = 32
C_ = 96
N_ = 224 * 224
CBLK = 8
WEIGHT_ = 1.0


def _lut_kernel(inp0_ref, mask_ref, style01t_ref, out_ref):
    x0 = inp0_ref[...] * mask_ref[...]
    mn0 = jnp.min(x0)
    mx0 = jnp.max(x0)
    step0 = (mx0 - mn0) / BINS_

    st = style01t_ref[...]
    rs = jnp.sum(st, axis=0, keepdims=True)
    sh = st * (float(N_) / rs)
    r = lax.broadcasted_iota(jnp.int32, (BINS_, BINS_), 0)
    c = lax.broadcasted_iota(jnp.int32, (BINS_, BINS_), 1)
    tri_lo = jnp.where(c <= r, 1.0, 0.0).astype(jnp.float32)
    cdfT = jnp.dot(tri_lo, sh, preferred_element_type=jnp.float32)
    cdf0 = cdfT[:, 0:1]
    cdf1_0 = cdfT[0:1, 1:2]

    m1 = lax.broadcasted_iota(jnp.int32, (1, 64), 1).astype(jnp.float32) + 1.0
    idx0 = jnp.sum(jnp.where(cdf0 < m1, 1.0, 0.0), axis=0, keepdims=True)

    jcol = lax.broadcasted_iota(jnp.int32, (64, 1), 0).astype(jnp.float32)
    eq = jnp.where(jcol == idx0, 1.0, 0.0)
    zs = jnp.zeros((31, 1), jnp.float32)
    z1 = jnp.zeros((1, 1), jnp.float32)
    cdfp_ext = jnp.concatenate([z1, cdf0[0:31], z1, zs], axis=0)
    cdf_ext = jnp.concatenate([cdf0, cdf1_0, zs], axis=0)
    cdfp_sel = jnp.sum(eq * cdfp_ext, axis=0, keepdims=True)
    cdf_sel = jnp.sum(eq * cdf_ext, axis=0, keepdims=True)

    ratio = jnp.clip((m1 - cdfp_sel) / (1e-8 + cdf_sel), 0.0, 1.0)
    lut = mn0 + (ratio + idx0) * step0

    dlut = lut[:, 1:33] - lut[:, 0:32]
    out_ref[...] = jnp.concatenate(
        [dlut, lut[:, 0:1], jnp.zeros((1, 31), jnp.float32)], axis=1)


def _loss_kernel(inp_ref, mask_ref, style_ref, lut_ref, out_ref, acc_ref):
    i = pl.program_id(0)
    nblk = pl.num_programs(0)

    st = style_ref[...]
    rs = jnp.sum(st, axis=1, keepdims=True)
    sh = st * (float(N_) / rs)
    r = lax.broadcasted_iota(jnp.int32, (BINS_, BINS_), 0)
    c = lax.broadcasted_iota(jnp.int32, (BINS_, BINS_), 1)
    tri_up = jnp.where(r <= c, 1.0, 0.0).astype(jnp.float32)
    cdf = jnp.dot(sh, tri_up, preferred_element_type=jnp.float32)

    x = inp_ref[...] * mask_ref[...]
    n1 = lax.broadcasted_iota(jnp.int32, (CBLK, N_), 1).astype(jnp.float32) + 1.0

    lut = lut_ref[...]
    lutv = jnp.zeros((CBLK, N_), jnp.float32) + lut[0:1, 32:33]
    for b in range(BINS_):
        lutv = lutv + jnp.where(n1 > cdf[:, b:b + 1], lut[0:1, b:b + 1], 0.0)

    diff = lutv - x
    part = jnp.sum(diff * diff)

    @pl.when(i == 0)
    def _():
        acc_ref[0] = part
        acc_ref[1] = jnp.sum(mask_ref[...])

    @pl.when(i > 0)
    def _():
        acc_ref[0] = acc_ref[0] + part

    @pl.when(i == nblk - 1)
    def _():
        total = float(C_) * float(N_)
        scale = (float(C_) / (total * total)) * WEIGHT_
        out_ref[...] = jnp.reshape((acc_ref[0] * scale) * acc_ref[1], (1, 1))


def kernel(input, mask_tight, mask_rough, style_his):
    inp2d = input.reshape(C_, N_)
    mask = mask_tight.reshape(1, N_)

    lut = pl.pallas_call(
        _lut_kernel,
        grid=(1,),
        in_specs=[
            pl.BlockSpec((1, N_), lambda i: (0, 0)),
            pl.BlockSpec((1, N_), lambda i: (0, 0)),
            pl.BlockSpec((BINS_, 2), lambda i: (0, 0)),
        ],
        out_specs=pl.BlockSpec((1, 64), lambda i: (0, 0)),
        out_shape=jax.ShapeDtypeStruct((1, 64), jnp.float32),
    )(inp2d[0:1], mask, style_his[0:2].T)

    out = pl.pallas_call(
        _loss_kernel,
        grid=(C_ // CBLK,),
        in_specs=[
            pl.BlockSpec((CBLK, N_), lambda i: (i, 0)),
            pl.BlockSpec((1, N_), lambda i: (0, 0)),
            pl.BlockSpec((CBLK, BINS_), lambda i: (i, 0)),
            pl.BlockSpec((1, 64), lambda i: (0, 0)),
        ],
        out_specs=pl.BlockSpec((1, 1), lambda i: (0, 0)),
        out_shape=jax.ShapeDtypeStruct((1, 1), jnp.float32),
        scratch_shapes=[pltpu.SMEM((2,), jnp.float32)],
    )(inp2d, mask, style_his, lut)

    return out[0, 0]


# speedup vs baseline: 2358.8001x; 2.8582x over previous
import jax
import jax.numpy as jnp
from jax import lax
from jax.experimental import pallas as pl
from jax.experimental.pallas import tpu as pltpu

BINS_ = 32
C_ = 96
N_ = 224 * 224
NT_ = N_ // 128
CBLK = 8
WEIGHT_ = 1.0


def _kernel(inp_ref, mask_ref, style_ref, out_ref, scr_ref, acc_ref):
    i = pl.program_id(0)
    nblk = pl.num_programs(0)
    f32 = jnp.float32

    st = style_ref[...]
    rs = jnp.sum(st, axis=1, keepdims=True)
    sh = st * (float(N_) / rs)
    r = lax.broadcasted_iota(jnp.int32, (BINS_, BINS_), 0)
    c = lax.broadcasted_iota(jnp.int32, (BINS_, BINS_), 1)
    tri_up = jnp.where(r <= c, 1.0, 0.0).astype(f32)
    cdf = jnp.dot(sh, tri_up, preferred_element_type=f32)
    Kf = jnp.clip(jnp.floor(cdf), 0.0, float(N_))
    Ki = Kf.astype(jnp.int32)
    t = jnp.minimum(Ki // 128, NT_ - 1)
    rem = Ki - t * 128
    q_part = jnp.sum(float(N_) - Kf, axis=0, keepdims=True)

    x = inp_ref[...] * mask_ref[...]
    x2d = jnp.reshape(x, (CBLK * NT_, 128))
    ones_col = jnp.ones((128, 1), f32)
    tsall = jnp.dot(x2d, ones_col, preferred_element_type=f32)
    ssall = jnp.dot(x2d * x2d, ones_col, preferred_element_type=f32)
    ss_part = jnp.sum(ssall)
    stot_part = jnp.sum(tsall)

    i392 = lax.broadcasted_iota(jnp.int32, (NT_, 1), 0)
    pos = lax.broadcasted_iota(jnp.int32, (128, 1), 0)
    t_row_acc = jnp.zeros((1, BINS_), f32)
    for cl in range(CBLK):
        xc = x[cl]
        tsc = tsall[cl * NT_:(cl + 1) * NT_]
        t_row = t[cl:cl + 1, :]
        rem_row = rem[cl:cl + 1, :]
        oh = jnp.where(i392 == t_row, 1.0, 0.0).astype(f32)
        selT = lax.dot_general(xc, oh, (((0,), (0,)), ((), ())),
                               preferred_element_type=f32)
        fine = jnp.sum(jnp.where(pos >= rem_row, selT, 0.0),
                       axis=0, keepdims=True)
        cmpgt = jnp.where(i392 > t_row, 1.0, 0.0).astype(f32)
        coarse = lax.dot_general(tsc, cmpgt, (((0,), (0,)), ((), ())),
                                 preferred_element_type=f32)
        t_row_acc = t_row_acc + coarse + fine

    @pl.when(i == 0)
    def _():
        scr_ref[0:1, :] = t_row_acc
        scr_ref[1:2, :] = q_part
        scr_ref[2:3, :] = cdf[0:1, :]
        acc_ref[0] = ss_part
        acc_ref[1] = jnp.sum(mask_ref[...])
        acc_ref[2] = jnp.min(x[0])
        acc_ref[3] = jnp.max(x[0])
        acc_ref[4] = stot_part
        acc_ref[5] = jnp.sum(cdf[1:2, 0:1])

    @pl.when(i > 0)
    def _():
        scr_ref[0:1, :] = scr_ref[0:1, :] + t_row_acc
        scr_ref[1:2, :] = scr_ref[1:2, :] + q_part
        acc_ref[0] = acc_ref[0] + ss_part
        acc_ref[4] = acc_ref[4] + stot_part

    @pl.when(i == nblk - 1)
    def _():
        mn0 = acc_ref[2]
        mx0 = acc_ref[3]
        step0 = (mx0 - mn0) / BINS_
        cdf0_row = scr_ref[2:3, :]
        cdf1_0 = jnp.reshape(acc_ref[5], (1, 1))
        m1 = lax.broadcasted_iota(jnp.int32, (64, 1), 0).astype(f32) + 1.0
        idx0 = jnp.sum(jnp.where(cdf0_row < m1, 1.0, 0.0),
                       axis=1, keepdims=True)
        jrow = lax.broadcasted_iota(jnp.int32, (1, 64), 1).astype(f32)
        eq = jnp.where(idx0 == jrow, 1.0, 0.0)
        z1 = jnp.zeros((1, 1), f32)
        cdfp_ext = jnp.concatenate(
            [z1, cdf0_row[:, 0:31], jnp.zeros((1, 32), f32)], axis=1)
        cdf_ext = jnp.concatenate(
            [cdf0_row, cdf1_0, jnp.zeros((1, 31), f32)], axis=1)
        cdfp_sel = jnp.sum(eq * cdfp_ext, axis=1, keepdims=True)
        cdf_sel = jnp.sum(eq * cdf_ext, axis=1, keepdims=True)
        ratio = jnp.clip((m1 - cdfp_sel) / (1e-8 + cdf_sel), 0.0, 1.0)
        lut = mn0 + (ratio + idx0) * step0

        dlut = lut[1:33] - lut[0:32]
        lutsq = lut * lut
        dlut2 = lutsq[1:33] - lutsq[0:32]
        lut0 = lut[0:1, 0:1]

        cross = jnp.dot(scr_ref[0:1, :], dlut,
                        preferred_element_type=f32)
        lut2t = jnp.dot(scr_ref[1:2, :], dlut2,
                        preferred_element_type=f32)
        total = float(C_) * float(N_)
        loss_sum = (acc_ref[0] - 2.0 * (lut0 * acc_ref[4] + cross)
                    + (total * lut0 * lut0 + lut2t))
        scale = (float(C_) / (total * total)) * WEIGHT_
        out_ref[...] = (loss_sum * scale) * acc_ref[1]


def kernel(input, mask_tight, mask_rough, style_his):
    inp3 = input.reshape(C_, NT_, 128)
    mask3 = mask_tight.reshape(1, NT_, 128)

    out = pl.pallas_call(
        _kernel,
        grid=(C_ // CBLK,),
        in_specs=[
            pl.BlockSpec((CBLK, NT_, 128), lambda i: (i, 0, 0)),
            pl.BlockSpec((1, NT_, 128), lambda i: (0, 0, 0)),
            pl.BlockSpec((CBLK, BINS_), lambda i: (i, 0)),
        ],
        out_specs=pl.BlockSpec((1, 1), lambda i: (0, 0)),
        out_shape=jax.ShapeDtypeStruct((1, 1), jnp.float32),
        scratch_shapes=[
            pltpu.VMEM((8, BINS_), jnp.float32),
            pltpu.SMEM((8,), jnp.float32),
        ],
    )(inp3, mask3, style_his)

    return out[0, 0]
